# baseline (speedup 1.0000x reference)
import functools

import jax
import jax.numpy as jnp
from jax import lax
from jax.experimental import pallas as pl
from jax.experimental.pallas import tpu as pltpu
from jax.experimental.pallas import tpu_sc as plsc

B = 16384
DIM = 32
PAIRS = DIM // 2
NC, NS = 2, 16
NW = NC * NS
B_PER_W = B // NW
E_PER_W = B_PER_W * PAIRS
MLP_BLOCK = 4096

UV = 1000001
RV = 100001
UPT = 7813
RPT = 782
UK = 601
RK = 391

PERM_LO = [16 * (i // 8) + i % 8 for i in range(PAIRS)]
PERM_HI = [16 * (i // 8) + i % 8 + 8 for i in range(PAIRS)]


def _flatten_pack_body(in_ref, out_ref):
    lo = in_ref[:8, :].astype(jnp.bfloat16)
    hi = in_ref[8:, :].astype(jnp.bfloat16)
    lo_u = lax.bitcast_convert_type(lo, jnp.uint16).astype(jnp.uint32)
    hi_u = lax.bitcast_convert_type(hi, jnp.uint16).astype(jnp.uint32)
    packed = lax.bitcast_convert_type(lo_u | (hi_u << 16), jnp.float32)
    out_ref[...] = pltpu.einshape(
        "s(tc)->(ts)c", packed, c=128, assert_is_tile_preserving=True
    )


def _flatten_pack(tableT, pt, k):
    ngroups = DIM // 16
    out_rows = ngroups * 8 * pt
    return pl.pallas_call(
        _flatten_pack_body,
        grid=(ngroups, pt // k),
        in_specs=[pl.BlockSpec((16, k * 128), lambda g, j: (g, j))],
        out_specs=pl.BlockSpec((k * 8, 128), lambda g, j, _pt=pt, _k=k:
                               (g * (_pt // _k) + j, 0)),
        out_shape=jax.ShapeDtypeStruct((out_rows, 128), jnp.float32),
        compiler_params=pltpu.CompilerParams(
            dimension_semantics=("parallel", "arbitrary")),
    )(tableT)


def _sc_gather(flat, ids, pt):
    gk = jnp.array(
        [(i // 8) * (pt * 1024) + (i % 8) * 128 for i in range(PAIRS)],
        dtype=jnp.int32,
    )

    @functools.partial(
        pl.kernel,
        mesh=plsc.VectorSubcoreMesh(core_axis_name="c", subcore_axis_name="s"),
        out_type=jax.ShapeDtypeStruct((B * PAIRS,), jnp.float32),
        scratch_types=[
            pltpu.VMEM((B_PER_W,), jnp.int32),
            pltpu.VMEM((PAIRS,), jnp.int32),
            pltpu.VMEM((E_PER_W,), jnp.int32),
            pltpu.VMEM((E_PER_W,), jnp.float32),
            pltpu.SemaphoreType.DMA,
        ],
    )
    def k(f_hbm, ids_hbm, gk_hbm, o_hbm, ids_v, gk_v, idx_v, val_v, sem):
        wid = lax.axis_index("s") * NC + lax.axis_index("c")
        base = wid * B_PER_W
        pltpu.sync_copy(ids_hbm.at[pl.ds(base, B_PER_W)], ids_v)
        pltpu.sync_copy(gk_hbm, gk_v)
        gk_row = gk_v[...]

        @plsc.parallel_loop(0, B_PER_W, unroll=4)
        def _(j):
            vj = ids_v[pl.ds(j, 1)]
            vterm = ((vj >> 7) << 10) + (vj & 127)
            idx_v[pl.ds(j * PAIRS, PAIRS)] = gk_row + vterm

        pltpu.async_copy(f_hbm.at[idx_v], val_v, sem).wait()
        pltpu.sync_copy(val_v, o_hbm.at[pl.ds(wid * E_PER_W, E_PER_W)])

    return k(flat, ids, gk)


def _unpack(p_u32):
    lo = lax.bitcast_convert_type(p_u32 << 16, jnp.float32)
    hi = lax.bitcast_convert_type(p_u32 & jnp.uint32(0xFFFF0000), jnp.float32)
    return lo.astype(jnp.bfloat16), hi.astype(jnp.bfloat16)


def _mlp_body(u_ref, r_ref, w1ul_ref, w1uh_ref, w1rl_ref, w1rh_ref, b1_ref,
              w2_ref, b2_ref, w3_ref, b3_ref, o_ref):
    u_lo, u_hi = _unpack(lax.bitcast_convert_type(u_ref[...], jnp.uint32))
    r_lo, r_hi = _unpack(lax.bitcast_convert_type(r_ref[...], jnp.uint32))
    f32 = jnp.float32
    h = jnp.dot(u_lo, w1ul_ref[...], preferred_element_type=f32)
    h += jnp.dot(u_hi, w1uh_ref[...], preferred_element_type=f32)
    h += jnp.dot(r_lo, w1rl_ref[...], preferred_element_type=f32)
    h += jnp.dot(r_hi, w1rh_ref[...], preferred_element_type=f32)
    h = jnp.maximum(h + b1_ref[...], 0.0)
    h = jnp.dot(h, w2_ref[...], preferred_element_type=f32)
    h = jnp.maximum(h + b2_ref[...], 0.0)
    o_ref[...] = (
        jnp.dot(h, w3_ref[...], preferred_element_type=f32) + b3_ref[...]
    )


def _tc_mlp(u, r, W1, b1, W2, b2, W3, b3):
    bf16 = jnp.bfloat16
    w1u = W1[:DIM]
    w1r = W1[DIM:]
    w1ul = w1u[jnp.array(PERM_LO)].astype(bf16)
    w1uh = w1u[jnp.array(PERM_HI)].astype(bf16)
    w1rl = w1r[jnp.array(PERM_LO)].astype(bf16)
    w1rh = w1r[jnp.array(PERM_HI)].astype(bf16)
    b1r = b1.reshape(1, -1)
    b2r = b2.reshape(1, -1)
    b3r = b3.reshape(1, -1)
    const = lambda shape: pl.BlockSpec(shape, lambda i: (0, 0))
    return pl.pallas_call(
        _mlp_body,
        grid=(B // MLP_BLOCK,),
        in_specs=[
            pl.BlockSpec((MLP_BLOCK, PAIRS), lambda i: (i, 0)),
            pl.BlockSpec((MLP_BLOCK, PAIRS), lambda i: (i, 0)),
            const(w1ul.shape),
            const(w1uh.shape),
            const(w1rl.shape),
            const(w1rh.shape),
            const(b1r.shape),
            const(W2.shape),
            const(b2r.shape),
            const(W3.shape),
            const(b3r.shape),
        ],
        out_specs=pl.BlockSpec((MLP_BLOCK, 1), lambda i: (i, 0)),
        out_shape=jax.ShapeDtypeStruct((B, 1), jnp.float32),
        compiler_params=pltpu.CompilerParams(
            dimension_semantics=("parallel",)),
    )(u, r, w1ul, w1uh, w1rl, w1rh, b1r, W2, b2r, W3, b3r)


def kernel(user_id, recipe_id, user_table, recipe_table, W1, b1, W2, b2, W3, b3):
    rflat = _flatten_pack(recipe_table.T, RPT, RK).reshape(-1)
    ro = _sc_gather(rflat, recipe_id.astype(jnp.int32), RPT)
    user_tableT, _ = lax.optimization_barrier((user_table.T, rflat))
    uflat = _flatten_pack(user_tableT, UPT, UK).reshape(-1)
    uo = _sc_gather(uflat, user_id.astype(jnp.int32), UPT)
    u = uo.reshape(B, PAIRS)
    r = ro.reshape(B, PAIRS)
    return _tc_mlp(u, r, W1, b1, W2, b2, W3, b3)
